# baseline (speedup 1.0000x reference)
import jax
import jax.numpy as jnp
from jax import lax
from jax.experimental import pallas as pl
from jax.experimental.pallas import tpu as pltpu, tpu_sc as plsc

T_CACHE = 600
K_CUR = 300
DECAY = 0.6
IDX1_PAD = 384
IDX2_PAD = 640
CH = 120
N_PAD = 904
NC, NS = 2, 16
B_PER_W = 2


def _cmax_body(conf_ref, out_ref):
    out_ref[...] = jnp.max(conf_ref[...], axis=-1, keepdims=True)


def _rank_col(v_col, v_row, n):
    ii = jax.lax.broadcasted_iota(jnp.int32, (n, n), 0)
    jj = jax.lax.broadcasted_iota(jnp.int32, (n, n), 1)
    beats = (v_row > v_col) | ((v_row == v_col) & (jj < ii))
    return jnp.sum(beats.astype(jnp.int32), axis=1, keepdims=True)


def _selT(rank, n, n_out):
    p = jax.lax.broadcasted_iota(jnp.int32, (n, n_out), 1)
    return (rank == p).astype(jnp.float32)


def _idx_row(rank, n, n_pad, off, start=0):
    p = jax.lax.broadcasted_iota(jnp.int32, (n, n_pad), 1) + start
    ii = jax.lax.broadcasted_iota(jnp.int32, (n, n_pad), 0)
    return jnp.sum((rank == p).astype(jnp.int32) * ii, axis=0) + off


def _gatherT(selT, x):
    return jax.lax.dot_general(
        selT, x, dimension_numbers=(((0,), (0,)), ((), ())),
        precision=jax.lax.Precision.DEFAULT,
        preferred_element_type=jnp.float32)


def _rank_body(cmax_c_ref, cmax_r_ref, confs_c_ref, confs_r_ref,
               anc_ref, canc_ref,
               fused_a_ref, new_a_ref, new_c_ref, idx1_ref, idx2_ref):
    n = cmax_r_ref.shape[-1]
    off = pl.program_id(0) * n
    anc = anc_ref[0]

    rank1 = _rank_col(cmax_c_ref[0], cmax_r_ref[0], n)
    rank2 = _rank_col(confs_c_ref[0], confs_r_ref[0], n)

    fused_a_ref[0, :T_CACHE, :] = canc_ref[0]
    fused_a_ref[0, T_CACHE:, :] = _gatherT(_selT(rank1, n, K_CUR), anc)
    sel2T = _selT(rank2, n, T_CACHE)
    new_a_ref[0] = _gatherT(sel2T, anc)
    new_c_ref[0, 0] = jnp.sum(sel2T * confs_c_ref[0], axis=0)
    idx1_ref[0, 0] = _idx_row(rank1, n, IDX1_PAD, off)
    idx2_ref[0, 0] = _idx_row(rank2, n, IDX2_PAD, off)


def _sc_body(feat_hbm, cfeat_hbm, idx1_hbm, idx2_hbm,
             fused_hbm, newf_hbm, idx_v0, idx_v1, buf0, buf1,
             sem_l, sem_g0, sem_g1, sem_s0, sem_s1):
    wid = lax.axis_index("s") * NC + lax.axis_index("c")
    idx_vs = (idx_v0, idx_v1)
    bufs = (buf0, buf1)
    gsems = (sem_g0, sem_g1)
    ssems = (sem_s0, sem_s1)
    lin = []
    chunks = []
    for j in range(B_PER_W):
        b = wid * B_PER_W + j
        fbase = b * N_PAD
        nbase = b * T_CACHE
        pltpu.sync_copy(idx1_hbm.at[pl.ds(b * IDX1_PAD, IDX1_PAD)],
                        idx_vs[j].at[pl.ds(0, IDX1_PAD)])
        pltpu.sync_copy(idx2_hbm.at[pl.ds(b * IDX2_PAD, IDX2_PAD)],
                        idx_vs[j].at[pl.ds(IDX1_PAD, IDX2_PAD)])
        lin.append(pltpu.async_copy(
            cfeat_hbm.at[pl.ds(nbase, T_CACHE)],
            fused_hbm.at[pl.ds(fbase, T_CACHE)], sem_l))
        for s, l in ((0, CH), (CH, CH), (2 * CH, 64)):
            chunks.append((s, j, fused_hbm, fbase + T_CACHE + s, l))
        for s in range(0, T_CACHE, CH):
            chunks.append((IDX1_PAD + s, j, newf_hbm, nbase + s, CH))

    def gather(k):
        s, j, _, _, l = chunks[k]
        return pltpu.async_copy(feat_hbm.at[idx_vs[j].at[pl.ds(s, l)]],
                                bufs[k % 2].at[pl.ds(0, l)], gsems[k % 2])

    nch = len(chunks)
    g = [None, None]
    st = [None, None]
    g[0] = gather(0)
    for k in range(nch):
        if k + 1 < nch:
            if st[(k + 1) % 2] is not None:
                st[(k + 1) % 2].wait()
            g[(k + 1) % 2] = gather(k + 1)
        g[k % 2].wait()
        _, _, dest, drow, l = chunks[k]
        st[k % 2] = pltpu.async_copy(bufs[k % 2].at[pl.ds(0, l)],
                                     dest.at[pl.ds(drow, l)], ssems[k % 2])
    for h in st + lin:
        if h is not None:
            h.wait()


def kernel(instance_feature, anchor, confidence, cached_feature,
           cached_anchor, cached_confidence):
    b, n, d = instance_feature.shape
    a = anchor.shape[-1]
    c = confidence.shape[-1]
    f32 = jnp.float32

    cmax = pl.pallas_call(
        _cmax_body,
        out_shape=jax.ShapeDtypeStruct((b * n, 1), f32),
    )(confidence.reshape(b * n, c)).reshape(b, n)

    sig = jax.nn.sigmoid(cmax)
    conf_s = jnp.concatenate(
        [jnp.maximum(cached_confidence * DECAY, sig[:, :T_CACHE]),
         sig[:, T_CACHE:]], axis=1)

    col = pl.BlockSpec((1, n, 1), lambda i: (i, 0, 0))
    row = pl.BlockSpec((1, 1, n), lambda i: (i, 0, 0))
    fused_a, new_a, new_c, idx1, idx2 = pl.pallas_call(
        _rank_body,
        grid=(b,),
        in_specs=[
            col, row, col, row,
            pl.BlockSpec((1, n, a), lambda i: (i, 0, 0)),
            pl.BlockSpec((1, T_CACHE, a), lambda i: (i, 0, 0)),
        ],
        out_specs=[
            pl.BlockSpec((1, n, a), lambda i: (i, 0, 0)),
            pl.BlockSpec((1, T_CACHE, a), lambda i: (i, 0, 0)),
            pl.BlockSpec((1, 1, T_CACHE), lambda i: (i, 0, 0)),
            pl.BlockSpec((1, 1, IDX1_PAD), lambda i: (i, 0, 0)),
            pl.BlockSpec((1, 1, IDX2_PAD), lambda i: (i, 0, 0)),
        ],
        out_shape=[
            jax.ShapeDtypeStruct((b, n, a), f32),
            jax.ShapeDtypeStruct((b, T_CACHE, a), f32),
            jax.ShapeDtypeStruct((b, 1, T_CACHE), f32),
            jax.ShapeDtypeStruct((b, 1, IDX1_PAD), jnp.int32),
            jax.ShapeDtypeStruct((b, 1, IDX2_PAD), jnp.int32),
        ],
    )(cmax.reshape(b, n, 1), cmax.reshape(b, 1, n),
      conf_s.reshape(b, n, 1), conf_s.reshape(b, 1, n),
      anchor, cached_anchor)

    mesh = plsc.VectorSubcoreMesh(core_axis_name="c", subcore_axis_name="s")
    fused_f_flat, new_f_flat = pl.kernel(
        _sc_body,
        out_type=[jax.ShapeDtypeStruct((b * N_PAD, d), f32),
                  jax.ShapeDtypeStruct((b * T_CACHE, d), f32)],
        mesh=mesh,
        scratch_types=[
            pltpu.VMEM((IDX1_PAD + IDX2_PAD,), jnp.int32),
            pltpu.VMEM((IDX1_PAD + IDX2_PAD,), jnp.int32),
            pltpu.VMEM((CH, d), f32),
            pltpu.VMEM((CH, d), f32),
            pltpu.SemaphoreType.DMA,
            pltpu.SemaphoreType.DMA,
            pltpu.SemaphoreType.DMA,
            pltpu.SemaphoreType.DMA,
            pltpu.SemaphoreType.DMA,
        ],
    )(instance_feature.reshape(b * n, d),
      cached_feature.reshape(b * T_CACHE, d),
      idx1.reshape(b * IDX1_PAD), idx2.reshape(b * IDX2_PAD))

    return (fused_f_flat.reshape(b, N_PAD, d)[:, :n, :], fused_a,
            new_f_flat.reshape(b, T_CACHE, d), new_a,
            new_c.reshape(b, T_CACHE))


# speedup vs baseline: 3.0861x; 3.0861x over previous
import jax
import jax.numpy as jnp
from jax import lax
from jax.experimental import pallas as pl
from jax.experimental.pallas import tpu as pltpu, tpu_sc as plsc

T_CACHE = 600
K_CUR = 300
DECAY = 0.6
IDX1_PAD = 384
IDX2_PAD = 640
CH = 120
N_PAD = 904
NC, NS = 2, 16
B_PER_W = 2


def _cmax_body(conf_ref, out_ref):
    out_ref[...] = jnp.max(conf_ref[...], axis=-1, keepdims=True)


def _rank_col(v_col, v_row, n):
    ii = jax.lax.broadcasted_iota(jnp.int32, (n, n), 0)
    jj = jax.lax.broadcasted_iota(jnp.int32, (n, n), 1)
    beats = (v_row > v_col) | ((v_row == v_col) & (jj < ii))
    return jnp.sum(beats.astype(jnp.int32), axis=1, keepdims=True)


def _selT(rank, n, n_out):
    p = jax.lax.broadcasted_iota(jnp.int32, (n, n_out), 1)
    return (rank == p).astype(jnp.float32)


def _idx_row(rank, n, n_pad, off, start=0):
    p = jax.lax.broadcasted_iota(jnp.int32, (n, n_pad), 1) + start
    ii = jax.lax.broadcasted_iota(jnp.int32, (n, n_pad), 0)
    return jnp.sum((rank == p).astype(jnp.int32) * ii, axis=0) + off


def _gatherT(selT, x):
    return jax.lax.dot_general(
        selT, x, dimension_numbers=(((0,), (0,)), ((), ())),
        precision=jax.lax.Precision.DEFAULT,
        preferred_element_type=jnp.float32)


def _rank_body(cmax_c_ref, cmax_r_ref, confs_c_ref, confs_r_ref,
               anc_ref, canc_ref,
               fused_a_ref, new_a_ref, new_c_ref, idx1_ref, idx2_ref):
    n = cmax_r_ref.shape[-1]
    off = pl.program_id(0) * n
    anc = anc_ref[0]

    rank1 = _rank_col(cmax_c_ref[0], cmax_r_ref[0], n)
    rank2 = _rank_col(confs_c_ref[0], confs_r_ref[0], n)

    fused_a_ref[0, :T_CACHE, :] = canc_ref[0]
    fused_a_ref[0, T_CACHE:, :] = _gatherT(_selT(rank1, n, K_CUR), anc)
    sel2T = _selT(rank2, n, T_CACHE)
    new_a_ref[0] = _gatherT(sel2T, anc)
    new_c_ref[0, 0] = jnp.sum(sel2T * confs_c_ref[0], axis=0)
    idx1_ref[0, 0] = _idx_row(rank1, n, IDX1_PAD, off)
    idx2_ref[0, 0] = _idx_row(rank2, n, IDX2_PAD, off)


def _sc_body(feat_hbm, cfeat_hbm, idx1_hbm, idx2_hbm,
             fused_hbm, newf_hbm, idx_v0, idx_v1, buf0, buf1, cbuf,
             sem_g0, sem_g1):
    wid = lax.axis_index("s") * NC + lax.axis_index("c")
    idx_vs = (idx_v0, idx_v1)
    bufs = (buf0, buf1)
    gsems = (sem_g0, sem_g1)
    chunks = []
    for j in range(B_PER_W):
        b = wid * B_PER_W + j
        fbase = b * N_PAD
        nbase = b * T_CACHE
        pltpu.sync_copy(idx1_hbm.at[pl.ds(b * IDX1_PAD, IDX1_PAD)],
                        idx_vs[j].at[pl.ds(0, IDX1_PAD)])
        pltpu.sync_copy(idx2_hbm.at[pl.ds(b * IDX2_PAD, IDX2_PAD)],
                        idx_vs[j].at[pl.ds(IDX1_PAD, IDX2_PAD)])
        for s, l in ((0, CH), (CH, CH), (2 * CH, 64)):
            chunks.append((s, j, fused_hbm, fbase + T_CACHE + s, l))
        for s in range(0, T_CACHE, CH):
            chunks.append((IDX1_PAD + s, j, newf_hbm, nbase + s, CH))

    def gather(k):
        s, j, _, _, l = chunks[k]
        return pltpu.async_copy(feat_hbm.at[idx_vs[j].at[pl.ds(s, l)]],
                                bufs[k % 2].at[pl.ds(0, l)], gsems[k % 2])

    nch = len(chunks)
    g = [gather(0), gather(1)]
    for j in range(B_PER_W):
        b = wid * B_PER_W + j
        for s in range(0, T_CACHE, CH):
            pltpu.sync_copy(cfeat_hbm.at[pl.ds(b * T_CACHE + s, CH)], cbuf)
            pltpu.sync_copy(cbuf, fused_hbm.at[pl.ds(b * N_PAD + s, CH)])
    for k in range(nch):
        g[k % 2].wait()
        _, _, dest, drow, l = chunks[k]
        pltpu.sync_copy(bufs[k % 2].at[pl.ds(0, l)],
                        dest.at[pl.ds(drow, l)])
        if k + 2 < nch:
            g[k % 2] = gather(k + 2)


def kernel(instance_feature, anchor, confidence, cached_feature,
           cached_anchor, cached_confidence):
    b, n, d = instance_feature.shape
    a = anchor.shape[-1]
    c = confidence.shape[-1]
    f32 = jnp.float32

    cmax = pl.pallas_call(
        _cmax_body,
        out_shape=jax.ShapeDtypeStruct((b * n, 1), f32),
    )(confidence.reshape(b * n, c)).reshape(b, n)

    sig = jax.nn.sigmoid(cmax)
    conf_s = jnp.concatenate(
        [jnp.maximum(cached_confidence * DECAY, sig[:, :T_CACHE]),
         sig[:, T_CACHE:]], axis=1)

    col = pl.BlockSpec((1, n, 1), lambda i: (i, 0, 0))
    row = pl.BlockSpec((1, 1, n), lambda i: (i, 0, 0))
    fused_a, new_a, new_c, idx1, idx2 = pl.pallas_call(
        _rank_body,
        grid=(b,),
        in_specs=[
            col, row, col, row,
            pl.BlockSpec((1, n, a), lambda i: (i, 0, 0)),
            pl.BlockSpec((1, T_CACHE, a), lambda i: (i, 0, 0)),
        ],
        out_specs=[
            pl.BlockSpec((1, n, a), lambda i: (i, 0, 0)),
            pl.BlockSpec((1, T_CACHE, a), lambda i: (i, 0, 0)),
            pl.BlockSpec((1, 1, T_CACHE), lambda i: (i, 0, 0)),
            pl.BlockSpec((1, 1, IDX1_PAD), lambda i: (i, 0, 0)),
            pl.BlockSpec((1, 1, IDX2_PAD), lambda i: (i, 0, 0)),
        ],
        out_shape=[
            jax.ShapeDtypeStruct((b, n, a), f32),
            jax.ShapeDtypeStruct((b, T_CACHE, a), f32),
            jax.ShapeDtypeStruct((b, 1, T_CACHE), f32),
            jax.ShapeDtypeStruct((b, 1, IDX1_PAD), jnp.int32),
            jax.ShapeDtypeStruct((b, 1, IDX2_PAD), jnp.int32),
        ],
    )(cmax.reshape(b, n, 1), cmax.reshape(b, 1, n),
      conf_s.reshape(b, n, 1), conf_s.reshape(b, 1, n),
      anchor, cached_anchor)

    mesh = plsc.VectorSubcoreMesh(core_axis_name="c", subcore_axis_name="s")
    fused_f_flat, new_f_flat = pl.kernel(
        _sc_body,
        out_type=[jax.ShapeDtypeStruct((b * N_PAD, d), f32),
                  jax.ShapeDtypeStruct((b * T_CACHE, d), f32)],
        mesh=mesh,
        scratch_types=[
            pltpu.VMEM((IDX1_PAD + IDX2_PAD,), jnp.int32),
            pltpu.VMEM((IDX1_PAD + IDX2_PAD,), jnp.int32),
            pltpu.VMEM((CH, d), f32),
            pltpu.VMEM((CH, d), f32),
            pltpu.VMEM((CH, d), f32),
            pltpu.SemaphoreType.DMA,
            pltpu.SemaphoreType.DMA,
        ],
    )(instance_feature.reshape(b * n, d),
      cached_feature.reshape(b * T_CACHE, d),
      idx1.reshape(b * IDX1_PAD), idx2.reshape(b * IDX2_PAD))

    return (fused_f_flat.reshape(b, N_PAD, d)[:, :n, :], fused_a,
            new_f_flat.reshape(b, T_CACHE, d), new_a,
            new_c.reshape(b, T_CACHE))
